# Initial kernel scaffold; baseline (speedup 1.0000x reference)
import jax
import jax.numpy as jnp
from jax.experimental import pallas as pl


def kernel(queries, keys, values, edges):
    raise NotImplementedError("write your pallas kernel here")


# speedup vs baseline: 8.3534x; 8.3534x over previous
import functools
import math

import jax
import jax.numpy as jnp
from jax import lax
from jax.experimental import pallas as pl
from jax.experimental.pallas import tpu as pltpu
from jax.experimental.pallas import tpu_sc as plsc

NC = 2
NS = 16
L = 16
PW = 16
NH = 4


def _chunk(n, limit=128):
    for c in range(limit, 7, -1):
        if c % 8 == 0 and n % c == 0:
            return c
    raise ValueError(f"no chunk for {n}")


def _divisor(n, limit):
    for c in range(limit, 0, -1):
        if n % c == 0:
            return c
    return 1


@functools.partial(jax.jit, static_argnames=("l", "hd", "n_edges"))
def _scores(qt, kt, iq, ik, esrc, *, l, hd, n_edges):
    temp = 1.0 / math.sqrt(hd // NH)
    hh2 = hd // 2
    epw = n_edges // (NC * NS)
    c1 = _chunk(epw)
    n_chunks = epw // c1
    rpt = -(-l // NS // 8) * 8
    lp = NS * rpt
    zrows = _divisor(rpt, 128)

    mesh = plsc.VectorSubcoreMesh(core_axis_name="c", subcore_axis_name="s",
                                  num_cores=NC, num_subcores=NS)

    @functools.partial(
        pl.kernel,
        out_type=(
            jax.ShapeDtypeStruct((n_edges, PW), jnp.float32),
            jax.ShapeDtypeStruct((NC, lp, hh2), jnp.float32),
        ),
        mesh=mesh,
        compiler_params=pltpu.CompilerParams(needs_layout_passes=False),
        scratch_types=[
            pltpu.VMEM((2 * c1, hh2), jnp.float32),
            pltpu.VMEM((2 * c1, hh2), jnp.float32),
            pltpu.VMEM((c1, PW), jnp.float32),
            pltpu.VMEM((c1, hh2), jnp.float32),
            pltpu.VMEM((L, L), jnp.float32),
            pltpu.VMEM((c1,), jnp.int32),
            pltpu.VMEM((2 * c1,), jnp.int32),
            pltpu.VMEM((2 * c1,), jnp.int32),
            pltpu.VMEM((8, hh2), jnp.float32),
            pltpu.VMEM_SHARED((lp, hh2), jnp.float32),
            pltpu.SemaphoreType.DMA,
            pltpu.SemaphoreType.DMA,
        ],
    )
    def k1(qt_hbm, kt_hbm, iq_hbm, ik_hbm, esrc_hbm, p_hbm, dpart_hbm,
           qbuf, kbuf, pbuf, pb128, ttr, isrc, iqb, ikb, zbuf, den_sp,
           qsem, ksem):
        cid = lax.axis_index("c")
        sid = lax.axis_index("s")
        wid = cid * NS + sid
        lanes = lax.iota(jnp.int32, L)
        epair = lanes >> 1
        hpair = lanes & 1
        pcol = [jnp.full((L,), p, jnp.int32) for p in range(16)]

        def zero_z(i, _):
            for j in range(hh2 // L):
                zbuf[i, pl.ds(j * L, L)] = jnp.zeros((L,), jnp.float32)
            return 0
        lax.fori_loop(0, 8, zero_z, 0)
        row0 = sid * rpt

        def zcp(j, _):
            pltpu.sync_copy(zbuf, den_sp.at[pl.ds(row0 + j * 8, 8)])
            return 0
        lax.fori_loop(0, rpt // 8, zcp, 0)

        def zero_p(e, _):
            pbuf[e, :] = jnp.zeros((L,), jnp.float32)
            for j in range(hh2 // L):
                pb128[e, pl.ds(j * L, L)] = jnp.zeros((L,), jnp.float32)
            return 0
        lax.fori_loop(0, c1, zero_p, 0)
        plsc.subcore_barrier()

        def chunk_body(i, _):
            base = wid * epw + i * c1
            pltpu.sync_copy(esrc_hbm.at[pl.ds(base, c1)], isrc)
            pltpu.sync_copy(iq_hbm.at[pl.ds(2 * base, 2 * c1)], iqb)
            pltpu.sync_copy(ik_hbm.at[pl.ds(2 * base, 2 * c1)], ikb)
            cpq = pltpu.async_copy(qt_hbm.at[iqb], qbuf, qsem)
            cpk = pltpu.async_copy(kt_hbm.at[ikb], kbuf, ksem)
            cpq.wait()
            cpk.wait()

            def blk_body(b, _):
                for g in range(NH // 2):
                    for pp in range(16):
                        el = b * 8 + pp // 2
                        hh = 2 * g + (pp % 2)
                        r = 2 * el + hh // 2
                        o = (hh % 2) * (hd // NH)
                        acc = qbuf[r, pl.ds(o, L)] * kbuf[r, pl.ds(o, L)]
                        for j in range(1, hd // NH // L):
                            acc = acc + (qbuf[r, pl.ds(o + j * L, L)]
                                         * kbuf[r, pl.ds(o + j * L, L)])
                        plsc.store_scatter(ttr, [lanes, pcol[pp]], acc)
                    s = ttr[0, :]
                    for j in range(1, L):
                        s = s + ttr[j, :]
                    pv = jnp.exp(s * temp)
                    plsc.store_scatter(
                        pbuf, [b * 8 + epair, 2 * g + hpair], pv)
                    plsc.store_scatter(
                        pb128, [b * 8 + epair, 2 * g + hpair], pv)
                return 0
            lax.fori_loop(0, c1 // 8, blk_body, 0)

            pltpu.sync_copy(pbuf, p_hbm.at[pl.ds(base, c1)])
            pltpu.sync_copy(pb128, den_sp.at[isrc], add=True)
            return 0
        lax.fori_loop(0, n_chunks, chunk_body, 0)

        plsc.subcore_barrier()
        pltpu.sync_copy(den_sp.at[pl.ds(row0, rpt)],
                        dpart_hbm.at[cid, pl.ds(row0, rpt)])

    return k1(qt, kt, iq, ik, esrc)


@functools.partial(jax.jit, static_argnames=("l", "hd", "n_edges"))
def _alpha(p, d0, d1, esrc, *, l, hd, n_edges):
    epw = n_edges // (NC * NS)
    c1 = _chunk(epw)
    n_chunks = epw // c1
    rpt = -(-l // NS // 8) * 8
    lp = NS * rpt
    hh2 = hd // 2
    sb = 32
    n_sb = lp // sb

    mesh = plsc.VectorSubcoreMesh(core_axis_name="c", subcore_axis_name="s",
                                  num_cores=NC, num_subcores=NS)

    @functools.partial(
        pl.kernel,
        out_type=(
            jax.ShapeDtypeStruct((n_edges, PW), jnp.float32),
            jax.ShapeDtypeStruct((NC, lp * NH), jnp.float32),
        ),
        mesh=mesh,
        compiler_params=pltpu.CompilerParams(needs_layout_passes=False),
        scratch_types=[
            pltpu.VMEM((c1, PW), jnp.float32),
            pltpu.VMEM((c1, PW), jnp.float32),
            pltpu.VMEM((c1,), jnp.int32),
            pltpu.VMEM((sb, hh2), jnp.float32),
            pltpu.VMEM((sb, hh2), jnp.float32),
            pltpu.VMEM((sb * NH,), jnp.float32),
            pltpu.VMEM((lp * NH,), jnp.float32),
        ],
    )
    def k2(p_hbm, d0_hbm, d1_hbm, esrc_hbm, al_hbm, dtx_hbm,
           pbuf, albuf, isrc, stg0, stg1, cbuf, dt):
        cid = lax.axis_index("c")
        sid = lax.axis_index("s")
        wid = cid * NS + sid
        lanes = lax.iota(jnp.int32, L)
        qrow = lanes >> 2
        qcol = lanes & 3

        def zero_a(e, _):
            albuf[e, :] = jnp.zeros((L,), jnp.float32)
            return 0
        lax.fori_loop(0, c1, zero_a, 0)

        @pl.when(sid == 0)
        def _():
            def stage_body(b, _):
                pltpu.sync_copy(d0_hbm.at[pl.ds(b * sb, sb)], stg0)
                pltpu.sync_copy(d1_hbm.at[pl.ds(b * sb, sb)], stg1)

                def cpt(i, _):
                    r = i * 4
                    v = (plsc.load_gather(stg0, [r + qrow, qcol])
                         + plsc.load_gather(stg1, [r + qrow, qcol]))
                    cbuf[pl.ds(r * NH, L)] = v
                    return 0
                lax.fori_loop(0, sb // 4, cpt, 0)
                pltpu.sync_copy(
                    cbuf, dtx_hbm.at[cid, pl.ds(b * sb * NH, sb * NH)])
                return 0
            lax.fori_loop(0, n_sb, stage_body, 0)
        plsc.subcore_barrier()
        pltpu.sync_copy(dtx_hbm.at[cid], dt)

        def chunk_body(i, _):
            base = wid * epw + i * c1
            pltpu.sync_copy(esrc_hbm.at[pl.ds(base, c1)], isrc)
            pltpu.sync_copy(p_hbm.at[pl.ds(base, c1)], pbuf)

            def alpha_blk(b, _):
                el4 = b * 4 + qrow
                src4 = plsc.load_gather(isrc, [el4])
                dv = plsc.load_gather(dt, [src4 * NH + qcol])
                pv = plsc.load_gather(pbuf, [el4, qcol])
                plsc.store_scatter(albuf, [el4, qcol], pv / (dv + 1e-16))
                return 0
            lax.fori_loop(0, c1 // 4, alpha_blk, 0)

            pltpu.sync_copy(albuf, al_hbm.at[pl.ds(base, c1)])
            return 0
        lax.fori_loop(0, n_chunks, chunk_body, 0)

    return k2(p, d0, d1, esrc)[0]


@functools.partial(jax.jit, static_argnames=("l", "hd", "n_edges"))
def _spmm(vt, al, esrc, iv, *, l, hd, n_edges):
    hw = hd // NC
    ept = n_edges // NS
    c2 = _chunk(ept)
    n_chunks = ept // c2
    rpt = -(-l // NS // 8) * 8
    lp = NS * rpt
    hps = NH // NC

    mesh = plsc.VectorSubcoreMesh(core_axis_name="c", subcore_axis_name="s",
                                  num_cores=NC, num_subcores=NS)

    @functools.partial(
        pl.kernel,
        out_type=jax.ShapeDtypeStruct((lp, NC * hw), jnp.float32),
        mesh=mesh,
        compiler_params=pltpu.CompilerParams(needs_layout_passes=False),
        scratch_types=[
            pltpu.VMEM((c2, hw), jnp.float32),
            pltpu.VMEM((c2, hw), jnp.float32),
            pltpu.VMEM((c2, PW), jnp.float32),
            pltpu.VMEM((c2,), jnp.int32),
            pltpu.VMEM((c2,), jnp.int32),
            pltpu.VMEM((8, hw), jnp.float32),
            pltpu.VMEM_SHARED((lp, hw), jnp.float32),
            pltpu.SemaphoreType.DMA,
        ],
    )
    def k3(vt_hbm, al_hbm, esrc_hbm, iv_hbm, out_hbm,
           vbuf, cbuf, albuf, isrc, ivv, zbuf, out_sp, vsem):
        cid = lax.axis_index("c")
        sid = lax.axis_index("s")
        hv0 = cid * hps

        def zero_z(i, _):
            for j in range(hw // L):
                zbuf[i, pl.ds(j * L, L)] = jnp.zeros((L,), jnp.float32)
            return 0
        lax.fori_loop(0, 8, zero_z, 0)
        row0 = sid * rpt

        def zcp(j, _):
            pltpu.sync_copy(zbuf, out_sp.at[pl.ds(row0 + j * 8, 8)])
            return 0
        lax.fori_loop(0, rpt // 8, zcp, 0)
        plsc.subcore_barrier()

        idx0 = jnp.full((L,), hv0, jnp.int32)
        idx1 = idx0 + 1

        def chunk_body(i, _):
            base = sid * ept + i * c2
            pltpu.sync_copy(esrc_hbm.at[pl.ds(base, c2)], isrc)
            pltpu.sync_copy(
                iv_hbm.at[pl.ds(cid * n_edges + base, c2)], ivv)
            cpv = pltpu.async_copy(vt_hbm.at[ivv], vbuf, vsem)
            pltpu.sync_copy(al_hbm.at[pl.ds(base, c2)], albuf)
            cpv.wait()

            def edge_body(e, _):
                erow = jnp.full((L,), e, jnp.int32)
                a0 = plsc.load_gather(albuf, [erow, idx0])
                a1 = plsc.load_gather(albuf, [erow, idx1])
                for j in range(hw // L // 2):
                    cbuf[e, pl.ds(j * L, L)] = vbuf[e, pl.ds(j * L, L)] * a0
                for j in range(hw // L // 2, hw // L):
                    cbuf[e, pl.ds(j * L, L)] = vbuf[e, pl.ds(j * L, L)] * a1
                return 0
            lax.fori_loop(0, c2, edge_body, 0)

            pltpu.sync_copy(cbuf, out_sp.at[isrc], add=True)
            return 0
        lax.fori_loop(0, n_chunks, chunk_body, 0)

        plsc.subcore_barrier()
        pltpu.sync_copy(out_sp.at[pl.ds(row0, rpt)],
                        out_hbm.at[pl.ds(row0, rpt), pl.ds(cid * hw, hw)])

    return k3(vt, al, esrc, iv)


def kernel(queries, keys, values, edges):
    l, h, e = queries.shape
    hd = h * e
    n_edges = edges.shape[1]
    esrc = edges[0].astype(jnp.int32)
    edst = edges[1].astype(jnp.int32)
    qt = queries.reshape(l * 2, hd // 2)
    kt = keys.reshape(l * 2, hd // 2)
    vt = values.reshape(l * 2, hd // 2)
    iq = jnp.stack([esrc * 2, esrc * 2 + 1], axis=1).reshape(2 * n_edges)
    ik = jnp.stack([edst * 2, edst * 2 + 1], axis=1).reshape(2 * n_edges)
    iv = jnp.concatenate([edst * 2, edst * 2 + 1])
    p, dpart = _scores(qt, kt, iq, ik, esrc, l=l, hd=hd, n_edges=n_edges)
    al = _alpha(p, dpart[0], dpart[1], esrc, l=l, hd=hd, n_edges=n_edges)
    out = _spmm(vt, al, esrc, iv, l=l, hd=hd, n_edges=n_edges)
    return out[:l].reshape(l, h, e)


# speedup vs baseline: 8.7304x; 1.0451x over previous
import functools
import math

import jax
import jax.numpy as jnp
from jax import lax
from jax.experimental import pallas as pl
from jax.experimental.pallas import tpu as pltpu
from jax.experimental.pallas import tpu_sc as plsc

NC = 2
NS = 16
L = 16
PW = 16
NH = 4


def _chunk(n, limit=128):
    for c in range(limit, 7, -1):
        if c % 8 == 0 and n % c == 0:
            return c
    raise ValueError(f"no chunk for {n}")


def _divisor(n, limit):
    for c in range(limit, 0, -1):
        if n % c == 0:
            return c
    return 1


@functools.partial(jax.jit, static_argnames=("l", "hd", "n_edges"))
def _scores(qt, kt, iq, ik, esrc, *, l, hd, n_edges):
    temp = 1.0 / math.sqrt(hd // NH)
    hh2 = hd // 2
    epw = n_edges // (NC * NS)
    c1 = _chunk(epw)
    n_chunks = epw // c1
    rpt = -(-l // NS // 8) * 8
    lp = NS * rpt
    zrows = _divisor(rpt, 128)

    mesh = plsc.VectorSubcoreMesh(core_axis_name="c", subcore_axis_name="s",
                                  num_cores=NC, num_subcores=NS)

    @functools.partial(
        pl.kernel,
        out_type=(
            jax.ShapeDtypeStruct((n_edges, PW), jnp.float32),
            jax.ShapeDtypeStruct((NC, lp, hh2), jnp.float32),
        ),
        mesh=mesh,
        compiler_params=pltpu.CompilerParams(needs_layout_passes=False),
        scratch_types=[
            pltpu.VMEM((2 * c1, hh2), jnp.float32),
            pltpu.VMEM((2 * c1, hh2), jnp.float32),
            pltpu.VMEM((c1, PW), jnp.float32),
            pltpu.VMEM((c1, hh2), jnp.float32),
            pltpu.VMEM((L, L), jnp.float32),
            pltpu.VMEM((c1,), jnp.int32),
            pltpu.VMEM((2 * c1,), jnp.int32),
            pltpu.VMEM((2 * c1,), jnp.int32),
            pltpu.VMEM((8, hh2), jnp.float32),
            pltpu.VMEM_SHARED((lp, hh2), jnp.float32),
            pltpu.SemaphoreType.DMA,
            pltpu.SemaphoreType.DMA,
        ],
    )
    def k1(qt_hbm, kt_hbm, iq_hbm, ik_hbm, esrc_hbm, p_hbm, dpart_hbm,
           qbuf, kbuf, pbuf, pb128, ttr, isrc, iqb, ikb, zbuf, den_sp,
           qsem, ksem):
        cid = lax.axis_index("c")
        sid = lax.axis_index("s")
        wid = cid * NS + sid
        lanes = lax.iota(jnp.int32, L)
        epair = lanes >> 1
        hpair = lanes & 1
        pcol = [jnp.full((L,), p, jnp.int32) for p in range(16)]

        def zero_z(i, _):
            for j in range(hh2 // L):
                zbuf[i, pl.ds(j * L, L)] = jnp.zeros((L,), jnp.float32)
            return 0
        lax.fori_loop(0, 8, zero_z, 0)
        row0 = sid * rpt

        def zcp(j, _):
            pltpu.sync_copy(zbuf, den_sp.at[pl.ds(row0 + j * 8, 8)])
            return 0
        lax.fori_loop(0, rpt // 8, zcp, 0)

        def zero_p(e, _):
            pbuf[e, :] = jnp.zeros((L,), jnp.float32)
            for j in range(hh2 // L):
                pb128[e, pl.ds(j * L, L)] = jnp.zeros((L,), jnp.float32)
            return 0
        lax.fori_loop(0, c1, zero_p, 0)
        plsc.subcore_barrier()

        def chunk_body(i, _):
            base = wid * epw + i * c1
            pltpu.sync_copy(esrc_hbm.at[pl.ds(base, c1)], isrc)
            pltpu.sync_copy(iq_hbm.at[pl.ds(2 * base, 2 * c1)], iqb)
            pltpu.sync_copy(ik_hbm.at[pl.ds(2 * base, 2 * c1)], ikb)
            cpq = pltpu.async_copy(qt_hbm.at[iqb], qbuf, qsem)
            cpk = pltpu.async_copy(kt_hbm.at[ikb], kbuf, ksem)
            cpq.wait()
            cpk.wait()

            def blk_body(b, _):
                for g in range(NH // 2):
                    for pp in range(16):
                        el = b * 8 + pp // 2
                        hh = 2 * g + (pp % 2)
                        r = 2 * el + hh // 2
                        o = (hh % 2) * (hd // NH)
                        acc = qbuf[r, pl.ds(o, L)] * kbuf[r, pl.ds(o, L)]
                        for j in range(1, hd // NH // L):
                            acc = acc + (qbuf[r, pl.ds(o + j * L, L)]
                                         * kbuf[r, pl.ds(o + j * L, L)])
                        plsc.store_scatter(ttr, [lanes, pcol[pp]], acc)
                    s = ttr[0, :]
                    for j in range(1, L):
                        s = s + ttr[j, :]
                    pv = jnp.exp(s * temp)
                    plsc.store_scatter(
                        pbuf, [b * 8 + epair, 2 * g + hpair], pv)
                    plsc.store_scatter(
                        pb128, [b * 8 + epair, 2 * g + hpair], pv)
                return 0
            lax.fori_loop(0, c1 // 8, blk_body, 0)

            pltpu.sync_copy(pbuf, p_hbm.at[pl.ds(base, c1)])
            pltpu.sync_copy(pb128, den_sp.at[isrc], add=True)
            return 0
        lax.fori_loop(0, n_chunks, chunk_body, 0)

        plsc.subcore_barrier()
        pltpu.sync_copy(den_sp.at[pl.ds(row0, rpt)],
                        dpart_hbm.at[cid, pl.ds(row0, rpt)])

    return k1(qt, kt, iq, ik, esrc)


@functools.partial(jax.jit, static_argnames=("l", "hd", "n_edges"))
def _alpha(p, d0, d1, esrc, *, l, hd, n_edges):
    epw = n_edges // (NC * NS)
    c1 = _chunk(epw, limit=256)
    n_chunks = epw // c1
    rpt = -(-l // NS // 8) * 8
    lp = NS * rpt
    hh2 = hd // 2
    sb = 32
    n_sb = lp // sb

    mesh = plsc.VectorSubcoreMesh(core_axis_name="c", subcore_axis_name="s",
                                  num_cores=NC, num_subcores=NS)

    @functools.partial(
        pl.kernel,
        out_type=(
            jax.ShapeDtypeStruct((n_edges, PW), jnp.float32),
            jax.ShapeDtypeStruct((NC, lp * NH), jnp.float32),
        ),
        mesh=mesh,
        compiler_params=pltpu.CompilerParams(needs_layout_passes=False),
        scratch_types=[
            pltpu.VMEM((c1, PW), jnp.float32),
            pltpu.VMEM((c1, PW), jnp.float32),
            pltpu.VMEM((c1,), jnp.int32),
            pltpu.VMEM((sb, hh2), jnp.float32),
            pltpu.VMEM((sb, hh2), jnp.float32),
            pltpu.VMEM((sb * NH,), jnp.float32),
            pltpu.VMEM((lp * NH,), jnp.float32),
        ],
    )
    def k2(p_hbm, d0_hbm, d1_hbm, esrc_hbm, al_hbm, dtx_hbm,
           pbuf, albuf, isrc, stg0, stg1, cbuf, dt):
        cid = lax.axis_index("c")
        sid = lax.axis_index("s")
        wid = cid * NS + sid
        lanes = lax.iota(jnp.int32, L)
        qrow = lanes >> 2
        qcol = lanes & 3

        def zero_a(e, _):
            albuf[e, :] = jnp.zeros((L,), jnp.float32)
            return 0
        lax.fori_loop(0, c1, zero_a, 0)

        @pl.when(sid == 0)
        def _():
            def stage_body(b, _):
                pltpu.sync_copy(d0_hbm.at[pl.ds(b * sb, sb)], stg0)
                pltpu.sync_copy(d1_hbm.at[pl.ds(b * sb, sb)], stg1)

                def cpt(i, _):
                    r = i * 4
                    v = (plsc.load_gather(stg0, [r + qrow, qcol])
                         + plsc.load_gather(stg1, [r + qrow, qcol]))
                    cbuf[pl.ds(r * NH, L)] = v
                    return 0
                lax.fori_loop(0, sb // 4, cpt, 0)
                pltpu.sync_copy(
                    cbuf, dtx_hbm.at[cid, pl.ds(b * sb * NH, sb * NH)])
                return 0
            lax.fori_loop(0, n_sb, stage_body, 0)
        plsc.subcore_barrier()
        pltpu.sync_copy(dtx_hbm.at[cid], dt)

        def chunk_body(i, _):
            base = wid * epw + i * c1
            pltpu.sync_copy(esrc_hbm.at[pl.ds(base, c1)], isrc)
            pltpu.sync_copy(p_hbm.at[pl.ds(base, c1)], pbuf)

            def alpha_blk(b, _):
                el4 = b * 4 + qrow
                src4 = plsc.load_gather(isrc, [el4])
                dv = plsc.load_gather(dt, [src4 * NH + qcol])
                pv = plsc.load_gather(pbuf, [el4, qcol])
                plsc.store_scatter(albuf, [el4, qcol], pv / (dv + 1e-16))
                return 0
            lax.fori_loop(0, c1 // 4, alpha_blk, 0)

            pltpu.sync_copy(albuf, al_hbm.at[pl.ds(base, c1)])
            return 0
        lax.fori_loop(0, n_chunks, chunk_body, 0)

    return k2(p, d0, d1, esrc)[0]


@functools.partial(jax.jit, static_argnames=("l", "hd", "n_edges"))
def _spmm(vt, al, esrc, iv, *, l, hd, n_edges):
    hw = hd // NC
    ept = n_edges // NS
    c2 = _chunk(ept)
    n_chunks = ept // c2
    rpt = -(-l // NS // 8) * 8
    lp = NS * rpt
    hps = NH // NC

    mesh = plsc.VectorSubcoreMesh(core_axis_name="c", subcore_axis_name="s",
                                  num_cores=NC, num_subcores=NS)

    @functools.partial(
        pl.kernel,
        out_type=jax.ShapeDtypeStruct((lp, NC * hw), jnp.float32),
        mesh=mesh,
        compiler_params=pltpu.CompilerParams(needs_layout_passes=False),
        scratch_types=[
            pltpu.VMEM((c2, hw), jnp.float32),
            pltpu.VMEM((c2, hw), jnp.float32),
            pltpu.VMEM((c2, PW), jnp.float32),
            pltpu.VMEM((c2,), jnp.int32),
            pltpu.VMEM((c2,), jnp.int32),
            pltpu.VMEM((8, hw), jnp.float32),
            pltpu.VMEM_SHARED((lp, hw), jnp.float32),
            pltpu.SemaphoreType.DMA,
        ],
    )
    def k3(vt_hbm, al_hbm, esrc_hbm, iv_hbm, out_hbm,
           vbuf, cbuf, albuf, isrc, ivv, zbuf, out_sp, vsem):
        cid = lax.axis_index("c")
        sid = lax.axis_index("s")
        hv0 = cid * hps

        def zero_z(i, _):
            for j in range(hw // L):
                zbuf[i, pl.ds(j * L, L)] = jnp.zeros((L,), jnp.float32)
            return 0
        lax.fori_loop(0, 8, zero_z, 0)
        row0 = sid * rpt

        def zcp(j, _):
            pltpu.sync_copy(zbuf, out_sp.at[pl.ds(row0 + j * 8, 8)])
            return 0
        lax.fori_loop(0, rpt // 8, zcp, 0)
        plsc.subcore_barrier()

        idx0 = jnp.full((L,), hv0, jnp.int32)
        idx1 = idx0 + 1

        def chunk_body(i, _):
            base = sid * ept + i * c2
            pltpu.sync_copy(esrc_hbm.at[pl.ds(base, c2)], isrc)
            pltpu.sync_copy(
                iv_hbm.at[pl.ds(cid * n_edges + base, c2)], ivv)
            cpv = pltpu.async_copy(vt_hbm.at[ivv], vbuf, vsem)
            pltpu.sync_copy(al_hbm.at[pl.ds(base, c2)], albuf)
            cpv.wait()

            def edge_body(e, _):
                erow = jnp.full((L,), e, jnp.int32)
                a0 = plsc.load_gather(albuf, [erow, idx0])
                a1 = plsc.load_gather(albuf, [erow, idx1])
                for j in range(hw // L // 2):
                    cbuf[e, pl.ds(j * L, L)] = vbuf[e, pl.ds(j * L, L)] * a0
                for j in range(hw // L // 2, hw // L):
                    cbuf[e, pl.ds(j * L, L)] = vbuf[e, pl.ds(j * L, L)] * a1
                return 0
            lax.fori_loop(0, c2, edge_body, 0)

            pltpu.sync_copy(cbuf, out_sp.at[isrc], add=True)
            return 0
        lax.fori_loop(0, n_chunks, chunk_body, 0)

        plsc.subcore_barrier()
        pltpu.sync_copy(out_sp.at[pl.ds(row0, rpt)],
                        out_hbm.at[pl.ds(row0, rpt), pl.ds(cid * hw, hw)])

    return k3(vt, al, esrc, iv)


def kernel(queries, keys, values, edges):
    l, h, e = queries.shape
    hd = h * e
    n_edges = edges.shape[1]
    esrc = edges[0].astype(jnp.int32)
    edst = edges[1].astype(jnp.int32)
    qt = queries.reshape(l * 2, hd // 2)
    kt = keys.reshape(l * 2, hd // 2)
    vt = values.reshape(l * 2, hd // 2)
    iq = jnp.stack([esrc * 2, esrc * 2 + 1], axis=1).reshape(2 * n_edges)
    ik = jnp.stack([edst * 2, edst * 2 + 1], axis=1).reshape(2 * n_edges)
    iv = jnp.concatenate([edst * 2, edst * 2 + 1])
    p, dpart = _scores(qt, kt, iq, ik, esrc, l=l, hd=hd, n_edges=n_edges)
    al = _alpha(p, dpart[0], dpart[1], esrc, l=l, hd=hd, n_edges=n_edges)
    out = _spmm(vt, al, esrc, iv, l=l, hd=hd, n_edges=n_edges)
    return out[:l].reshape(l, h, e)


# speedup vs baseline: 10.8917x; 1.2476x over previous
import functools
import math

import jax
import jax.numpy as jnp
from jax import lax
from jax.experimental import pallas as pl
from jax.experimental.pallas import tpu as pltpu
from jax.experimental.pallas import tpu_sc as plsc

NC = 2
NS = 16
L = 16
PW = 16
NH = 4


def _chunk(n, limit=128):
    for c in range(limit, 7, -1):
        if c % 8 == 0 and n % c == 0:
            return c
    raise ValueError(f"no chunk for {n}")


def _divisor(n, limit):
    for c in range(limit, 0, -1):
        if n % c == 0:
            return c
    return 1


@functools.partial(jax.jit, static_argnames=("l", "hd", "n_edges"))
def _scores(qt, kt, iq, ik, esrc, *, l, hd, n_edges):
    temp = 1.0 / math.sqrt(hd // NH)
    hh2 = hd // 2
    epw = n_edges // (NC * NS)
    c1 = _chunk(epw)
    n_chunks = epw // c1
    rpt = -(-l // NS // 8) * 8
    lp = NS * rpt
    zrows = _divisor(rpt, 128)

    mesh = plsc.VectorSubcoreMesh(core_axis_name="c", subcore_axis_name="s",
                                  num_cores=NC, num_subcores=NS)

    @functools.partial(
        pl.kernel,
        out_type=(
            jax.ShapeDtypeStruct((n_edges, PW), jnp.float32),
            jax.ShapeDtypeStruct((NC, lp, hh2), jnp.float32),
        ),
        mesh=mesh,
        compiler_params=pltpu.CompilerParams(needs_layout_passes=False),
        scratch_types=[
            pltpu.VMEM((2 * c1, hh2), jnp.float32),
            pltpu.VMEM((2 * c1, hh2), jnp.float32),
            pltpu.VMEM((c1, PW), jnp.float32),
            pltpu.VMEM((c1, hh2), jnp.float32),
            pltpu.VMEM((L, L), jnp.float32),
            pltpu.VMEM((c1,), jnp.int32),
            pltpu.VMEM((2 * c1,), jnp.int32),
            pltpu.VMEM((2 * c1,), jnp.int32),
            pltpu.VMEM((8, hh2), jnp.float32),
            pltpu.VMEM_SHARED((lp, hh2), jnp.float32),
            pltpu.SemaphoreType.DMA,
            pltpu.SemaphoreType.DMA,
        ],
    )
    def k1(qt_hbm, kt_hbm, iq_hbm, ik_hbm, esrc_hbm, p_hbm, dpart_hbm,
           qbuf, kbuf, pbuf, pb128, ttr, isrc, iqb, ikb, zbuf, den_sp,
           qsem, ksem):
        cid = lax.axis_index("c")
        sid = lax.axis_index("s")
        wid = cid * NS + sid
        lanes = lax.iota(jnp.int32, L)
        epair = lanes >> 1
        hpair = lanes & 1
        pcol = [jnp.full((L,), p, jnp.int32) for p in range(16)]

        def zero_z(i, _):
            for j in range(hh2 // L):
                zbuf[i, pl.ds(j * L, L)] = jnp.zeros((L,), jnp.float32)
            return 0
        lax.fori_loop(0, 8, zero_z, 0)
        row0 = sid * rpt

        def zcp(j, _):
            pltpu.sync_copy(zbuf, den_sp.at[pl.ds(row0 + j * 8, 8)])
            return 0
        lax.fori_loop(0, rpt // 8, zcp, 0)

        def zero_p(e, _):
            pbuf[e, :] = jnp.zeros((L,), jnp.float32)
            for j in range(hh2 // L):
                pb128[e, pl.ds(j * L, L)] = jnp.zeros((L,), jnp.float32)
            return 0
        lax.fori_loop(0, c1, zero_p, 0)
        plsc.subcore_barrier()

        def chunk_body(i, _):
            base = wid * epw + i * c1
            pltpu.sync_copy(esrc_hbm.at[pl.ds(base, c1)], isrc)
            pltpu.sync_copy(iq_hbm.at[pl.ds(2 * base, 2 * c1)], iqb)
            pltpu.sync_copy(ik_hbm.at[pl.ds(2 * base, 2 * c1)], ikb)
            cpq = pltpu.async_copy(qt_hbm.at[iqb], qbuf, qsem)
            cpk = pltpu.async_copy(kt_hbm.at[ikb], kbuf, ksem)
            cpq.wait()
            cpk.wait()

            def blk_body(b, _):
                for g in range(NH // 2):
                    for pp in range(16):
                        el = b * 8 + pp // 2
                        hh = 2 * g + (pp % 2)
                        r = 2 * el + hh // 2
                        o = (hh % 2) * (hd // NH)
                        acc = qbuf[r, pl.ds(o, L)] * kbuf[r, pl.ds(o, L)]
                        for j in range(1, hd // NH // L):
                            acc = acc + (qbuf[r, pl.ds(o + j * L, L)]
                                         * kbuf[r, pl.ds(o + j * L, L)])
                        plsc.store_scatter(ttr, [lanes, pcol[pp]], acc)
                    s = ttr[0, :]
                    for j in range(1, L):
                        s = s + ttr[j, :]
                    pv = jnp.exp(s * temp)
                    plsc.store_scatter(
                        pbuf, [b * 8 + epair, 2 * g + hpair], pv)
                    plsc.store_scatter(
                        pb128, [b * 8 + epair, 2 * g + hpair], pv)
                return 0
            lax.fori_loop(0, c1 // 8, blk_body, 0)

            pltpu.sync_copy(pbuf, p_hbm.at[pl.ds(base, c1)])
            pltpu.sync_copy(pb128, den_sp.at[isrc], add=True)
            return 0
        lax.fori_loop(0, n_chunks, chunk_body, 0)

        plsc.subcore_barrier()
        pltpu.sync_copy(den_sp.at[pl.ds(row0, rpt)],
                        dpart_hbm.at[cid, pl.ds(row0, rpt)])

    return k1(qt, kt, iq, ik, esrc)


@functools.partial(jax.jit, static_argnames=("l", "hd", "n_edges"))
def _alpha(p, d0, d1, esrc, *, l, hd, n_edges):
    epw = n_edges // (NC * NS)
    c1 = _chunk(epw, limit=256)
    n_chunks = epw // c1
    rpt = -(-l // NS // 8) * 8
    lp = NS * rpt
    hh2 = hd // 2
    sb = 32
    n_sb = lp // sb

    mesh = plsc.VectorSubcoreMesh(core_axis_name="c", subcore_axis_name="s",
                                  num_cores=NC, num_subcores=NS)

    @functools.partial(
        pl.kernel,
        out_type=(
            jax.ShapeDtypeStruct((n_edges, PW), jnp.float32),
            jax.ShapeDtypeStruct((NC, lp * NH), jnp.float32),
        ),
        mesh=mesh,
        compiler_params=pltpu.CompilerParams(needs_layout_passes=False),
        scratch_types=[
            pltpu.VMEM((c1, PW), jnp.float32),
            pltpu.VMEM((c1, PW), jnp.float32),
            pltpu.VMEM((c1,), jnp.int32),
            pltpu.VMEM((sb, hh2), jnp.float32),
            pltpu.VMEM((sb, hh2), jnp.float32),
            pltpu.VMEM((sb * NH,), jnp.float32),
            pltpu.VMEM((lp * NH,), jnp.float32),
        ],
    )
    def k2(p_hbm, d0_hbm, d1_hbm, esrc_hbm, al_hbm, dtx_hbm,
           pbuf, albuf, isrc, stg0, stg1, cbuf, dt):
        cid = lax.axis_index("c")
        sid = lax.axis_index("s")
        wid = cid * NS + sid
        lanes = lax.iota(jnp.int32, L)
        qrow = lanes >> 2
        qcol = lanes & 3

        def zero_a(e, _):
            albuf[e, :] = jnp.zeros((L,), jnp.float32)
            return 0
        lax.fori_loop(0, c1, zero_a, 0)

        nb = (n_sb - sid + NS - 1) // NS

        def stage_body(j, _):
            b = sid + j * NS
            pltpu.sync_copy(d0_hbm.at[pl.ds(b * sb, sb)], stg0)
            pltpu.sync_copy(d1_hbm.at[pl.ds(b * sb, sb)], stg1)

            def cpt(i, _):
                r = i * 4
                v = (plsc.load_gather(stg0, [r + qrow, qcol])
                     + plsc.load_gather(stg1, [r + qrow, qcol]))
                cbuf[pl.ds(r * NH, L)] = v
                return 0
            lax.fori_loop(0, sb // 4, cpt, 0)
            pltpu.sync_copy(
                cbuf, dtx_hbm.at[cid, pl.ds(b * sb * NH, sb * NH)])
            return 0
        lax.fori_loop(0, nb, stage_body, 0)
        plsc.subcore_barrier()
        pltpu.sync_copy(dtx_hbm.at[cid], dt)

        def chunk_body(i, _):
            base = wid * epw + i * c1
            pltpu.sync_copy(esrc_hbm.at[pl.ds(base, c1)], isrc)
            pltpu.sync_copy(p_hbm.at[pl.ds(base, c1)], pbuf)

            def alpha_blk(b, _):
                el4 = b * 4 + qrow
                src4 = plsc.load_gather(isrc, [el4])
                dv = plsc.load_gather(dt, [src4 * NH + qcol])
                pv = plsc.load_gather(pbuf, [el4, qcol])
                plsc.store_scatter(albuf, [el4, qcol], pv / (dv + 1e-16))
                return 0
            lax.fori_loop(0, c1 // 4, alpha_blk, 0)

            pltpu.sync_copy(albuf, al_hbm.at[pl.ds(base, c1)])
            return 0
        lax.fori_loop(0, n_chunks, chunk_body, 0)

    return k2(p, d0, d1, esrc)[0]


@functools.partial(jax.jit, static_argnames=("l", "hd", "n_edges"))
def _spmm(vt, al, esrc, iv, *, l, hd, n_edges):
    hw = hd // NC
    ept = n_edges // NS
    c2 = _chunk(ept)
    n_chunks = ept // c2
    rpt = -(-l // NS // 8) * 8
    lp = NS * rpt
    hps = NH // NC

    mesh = plsc.VectorSubcoreMesh(core_axis_name="c", subcore_axis_name="s",
                                  num_cores=NC, num_subcores=NS)

    @functools.partial(
        pl.kernel,
        out_type=jax.ShapeDtypeStruct((lp, NC * hw), jnp.float32),
        mesh=mesh,
        compiler_params=pltpu.CompilerParams(needs_layout_passes=False),
        scratch_types=[
            pltpu.VMEM((c2, hw), jnp.float32),
            pltpu.VMEM((c2, hw), jnp.float32),
            pltpu.VMEM((c2, PW), jnp.float32),
            pltpu.VMEM((c2,), jnp.int32),
            pltpu.VMEM((c2,), jnp.int32),
            pltpu.VMEM((8, hw), jnp.float32),
            pltpu.VMEM_SHARED((lp, hw), jnp.float32),
            pltpu.SemaphoreType.DMA,
        ],
    )
    def k3(vt_hbm, al_hbm, esrc_hbm, iv_hbm, out_hbm,
           vbuf, cbuf, albuf, isrc, ivv, zbuf, out_sp, vsem):
        cid = lax.axis_index("c")
        sid = lax.axis_index("s")
        hv0 = cid * hps

        def zero_z(i, _):
            for j in range(hw // L):
                zbuf[i, pl.ds(j * L, L)] = jnp.zeros((L,), jnp.float32)
            return 0
        lax.fori_loop(0, 8, zero_z, 0)
        row0 = sid * rpt

        def zcp(j, _):
            pltpu.sync_copy(zbuf, out_sp.at[pl.ds(row0 + j * 8, 8)])
            return 0
        lax.fori_loop(0, rpt // 8, zcp, 0)
        plsc.subcore_barrier()

        idx0 = jnp.full((L,), hv0, jnp.int32)
        idx1 = idx0 + 1

        def chunk_body(i, _):
            base = sid * ept + i * c2
            pltpu.sync_copy(esrc_hbm.at[pl.ds(base, c2)], isrc)
            pltpu.sync_copy(
                iv_hbm.at[pl.ds(cid * n_edges + base, c2)], ivv)
            cpv = pltpu.async_copy(vt_hbm.at[ivv], vbuf, vsem)
            pltpu.sync_copy(al_hbm.at[pl.ds(base, c2)], albuf)
            cpv.wait()

            def edge_body(e, _):
                erow = jnp.full((L,), e, jnp.int32)
                a0 = plsc.load_gather(albuf, [erow, idx0])
                a1 = plsc.load_gather(albuf, [erow, idx1])
                for j in range(hw // L // 2):
                    cbuf[e, pl.ds(j * L, L)] = vbuf[e, pl.ds(j * L, L)] * a0
                for j in range(hw // L // 2, hw // L):
                    cbuf[e, pl.ds(j * L, L)] = vbuf[e, pl.ds(j * L, L)] * a1
                return 0
            lax.fori_loop(0, c2, edge_body, 0)

            pltpu.sync_copy(cbuf, out_sp.at[isrc], add=True)
            return 0
        lax.fori_loop(0, n_chunks, chunk_body, 0)

        plsc.subcore_barrier()
        pltpu.sync_copy(out_sp.at[pl.ds(row0, rpt)],
                        out_hbm.at[pl.ds(row0, rpt), pl.ds(cid * hw, hw)])

    return k3(vt, al, esrc, iv)


def kernel(queries, keys, values, edges):
    l, h, e = queries.shape
    hd = h * e
    n_edges = edges.shape[1]
    esrc = edges[0].astype(jnp.int32)
    edst = edges[1].astype(jnp.int32)
    qt = queries.reshape(l * 2, hd // 2)
    kt = keys.reshape(l * 2, hd // 2)
    vt = values.reshape(l * 2, hd // 2)
    iq = jnp.stack([esrc * 2, esrc * 2 + 1], axis=1).reshape(2 * n_edges)
    ik = jnp.stack([edst * 2, edst * 2 + 1], axis=1).reshape(2 * n_edges)
    iv = jnp.concatenate([edst * 2, edst * 2 + 1])
    p, dpart = _scores(qt, kt, iq, ik, esrc, l=l, hd=hd, n_edges=n_edges)
    al = _alpha(p, dpart[0], dpart[1], esrc, l=l, hd=hd, n_edges=n_edges)
    out = _spmm(vt, al, esrc, iv, l=l, hd=hd, n_edges=n_edges)
    return out[:l].reshape(l, h, e)
